# baseline (speedup 1.0000x reference)
import functools

import jax
import jax.numpy as jnp
from jax import lax
from jax.experimental import pallas as pl
from jax.experimental.pallas import tpu as pltpu
from jax.experimental.pallas import tpu_sc as plsc

N = 10000
E = 320000
D = 128

NC = 2
NS = 16
NW = NC * NS

CH = 128
C = 79
E_PAD = NW * C * CH
N_ACC = N + 112
RPT = N_ACC // NS

ROWS_BLK = 2000
NB = N // ROWS_BLK


def _mm_body(x_ref, awt_ref, ab_ref, bwt_ref, bb_ref, ax_ref, bx_ref):
    x = x_ref[...]
    ax_ref[...] = (
        jnp.dot(x, awt_ref[...], preferred_element_type=jnp.float32) + ab_ref[...]
    )
    bx_ref[...] = (
        jnp.dot(x, bwt_ref[...], preferred_element_type=jnp.float32) + bb_ref[...]
    )


def _matmuls(X, A_wT, A_b2, B_wT, B_b2):
    return pl.pallas_call(
        _mm_body,
        grid=(NB,),
        in_specs=[
            pl.BlockSpec((ROWS_BLK, D), lambda i: (i, 0)),
            pl.BlockSpec((D, D), lambda i: (0, 0)),
            pl.BlockSpec((1, D), lambda i: (0, 0)),
            pl.BlockSpec((D, D), lambda i: (0, 0)),
            pl.BlockSpec((1, D), lambda i: (0, 0)),
        ],
        out_specs=[
            pl.BlockSpec((ROWS_BLK, D), lambda i: (i, 0)),
            pl.BlockSpec((ROWS_BLK, D), lambda i: (i, 0)),
        ],
        out_shape=[
            jax.ShapeDtypeStruct((N, D), jnp.float32),
            jax.ShapeDtypeStruct((N, D), jnp.float32),
        ],
    )(X, A_wT, A_b2, B_wT, B_b2)


@functools.cache
def _make_sc_agg():
    @functools.partial(
        pl.kernel,
        out_type=jax.ShapeDtypeStruct((NC, N_ACC, D), jnp.float32),
        mesh=plsc.VectorSubcoreMesh(core_axis_name="c", subcore_axis_name="s"),
        scratch_types=[
            pltpu.VMEM((C, CH), jnp.int32),
            pltpu.VMEM((C, CH), jnp.int32),
            pltpu.VMEM((CH, D), jnp.float32),
            pltpu.VMEM_SHARED((N_ACC, D), jnp.float32),
            pltpu.SemaphoreType.DMA,
        ],
    )
    def _sc_agg(bx_hbm, src_hbm, dst_hbm, zeros_hbm, out_hbm,
                src_v, dst_v, rows_v, acc_sh, sem):
        cid = lax.axis_index("c")
        sid = lax.axis_index("s")
        r0 = sid * RPT
        pltpu.sync_copy(src_hbm.at[cid, sid], src_v)
        pltpu.sync_copy(dst_hbm.at[cid, sid], dst_v)
        pltpu.sync_copy(zeros_hbm.at[pl.ds(r0, RPT)], acc_sh.at[pl.ds(r0, RPT)])
        plsc.subcore_barrier()

        @pl.loop(0, C)
        def _(j):
            pltpu.async_copy(bx_hbm.at[src_v.at[j]], rows_v, sem).wait()
            pltpu.sync_copy(rows_v, acc_sh.at[dst_v.at[j]], add=True)

        plsc.subcore_barrier()
        pltpu.sync_copy(acc_sh.at[pl.ds(r0, RPT)], out_hbm.at[cid, pl.ds(r0, RPT)])

    return _sc_agg


def _post_body(ax_ref, p0_ref, p1_ref, sn_ref, x_ref, g_ref, b_ref,
               out_ref, sum_ref, sq_ref):
    ph = pl.program_id(0)
    i = pl.program_id(1)
    hpre = (ax_ref[...] + p0_ref[0] + p1_ref[0]) * sn_ref[...]

    @pl.when(jnp.logical_and(ph == 0, i == 0))
    def _():
        sum_ref[...] = jnp.zeros_like(sum_ref)
        sq_ref[...] = jnp.zeros_like(sq_ref)

    @pl.when(ph == 0)
    def _():
        sum_ref[...] += jnp.sum(hpre, axis=0, keepdims=True)
        sq_ref[...] += jnp.sum(hpre * hpre, axis=0, keepdims=True)

    @pl.when(ph == 1)
    def _():
        mean = sum_ref[...] * (1.0 / N)
        var = sq_ref[...] * (1.0 / N) - mean * mean
        scale = lax.rsqrt(var + 1e-5) * g_ref[...]
        h = (hpre - mean) * scale + b_ref[...]
        out_ref[...] = x_ref[...] + jnp.maximum(h, 0.0)


def _post(AX, parts, snorm_n, X, g2, b2):
    return pl.pallas_call(
        _post_body,
        grid=(2, NB),
        in_specs=[
            pl.BlockSpec((ROWS_BLK, D), lambda p, i: (i, 0)),
            pl.BlockSpec((1, ROWS_BLK, D), lambda p, i: (0, i, 0)),
            pl.BlockSpec((1, ROWS_BLK, D), lambda p, i: (1, i, 0)),
            pl.BlockSpec((ROWS_BLK, 1), lambda p, i: (i, 0)),
            pl.BlockSpec((ROWS_BLK, D), lambda p, i: (i, 0)),
            pl.BlockSpec((1, D), lambda p, i: (0, 0)),
            pl.BlockSpec((1, D), lambda p, i: (0, 0)),
        ],
        out_specs=pl.BlockSpec((ROWS_BLK, D), lambda p, i: (i, 0)),
        out_shape=jax.ShapeDtypeStruct((N, D), jnp.float32),
        scratch_shapes=[
            pltpu.VMEM((1, D), jnp.float32),
            pltpu.VMEM((1, D), jnp.float32),
        ],
    )(AX, parts, parts, snorm_n, X, g2, b2)


def kernel(X, edge_index, E_X, snorm_n, snorm_e, A_w, A_b, B_w, B_b, gamma_h, beta_h):
    src = edge_index[0]
    dst = edge_index[1]
    pad = E_PAD - E
    src_p = jnp.concatenate(
        [src, jnp.zeros((pad,), src.dtype)]).reshape(NC, NS, C, CH)
    trash = N + jnp.arange(pad, dtype=dst.dtype) % (N_ACC - N)
    dst_p = jnp.concatenate([dst, trash]).reshape(NC, NS, C, CH)
    zeros = jnp.zeros((N_ACC, D), jnp.float32)

    AX, BX = _matmuls(X, A_w.T, A_b.reshape(1, D), B_w.T, B_b.reshape(1, D))
    parts = _make_sc_agg()(BX, src_p, dst_p, zeros)
    H = _post(AX, parts, snorm_n, X,
              gamma_h.reshape(1, D), beta_h.reshape(1, D))
    return (H, E_X)


# speedup vs baseline: 1.1863x; 1.1863x over previous
import functools

import jax
import jax.numpy as jnp
from jax import lax
from jax.experimental import pallas as pl
from jax.experimental.pallas import tpu as pltpu
from jax.experimental.pallas import tpu_sc as plsc

N = 10000
E = 320000
D = 128

NC = 2
NS = 16
NW = NC * NS

CH = 128
C0 = 118
C1 = 40
CMAX = C0
E_PAD = NS * CH * (C0 + C1)
N_ACC = N + 112
RPT = N_ACC // NS

ROWS_BLK = 2000
NB = N // ROWS_BLK


def _mm_body(x_ref, awt_ref, ab_ref, bwt_ref, bb_ref, ax_ref, bx_ref):
    x = x_ref[...]
    ax_ref[...] = (
        jnp.dot(x, awt_ref[...], preferred_element_type=jnp.float32) + ab_ref[...]
    )
    bx_ref[...] = (
        jnp.dot(x, bwt_ref[...], preferred_element_type=jnp.float32) + bb_ref[...]
    )


def _matmuls(X, A_wT, A_b2, B_wT, B_b2):
    return pl.pallas_call(
        _mm_body,
        grid=(NB,),
        in_specs=[
            pl.BlockSpec((ROWS_BLK, D), lambda i: (i, 0)),
            pl.BlockSpec((D, D), lambda i: (0, 0)),
            pl.BlockSpec((1, D), lambda i: (0, 0)),
            pl.BlockSpec((D, D), lambda i: (0, 0)),
            pl.BlockSpec((1, D), lambda i: (0, 0)),
        ],
        out_specs=[
            pl.BlockSpec((ROWS_BLK, D), lambda i: (i, 0)),
            pl.BlockSpec((ROWS_BLK, D), lambda i: (i, 0)),
        ],
        out_shape=[
            jax.ShapeDtypeStruct((N, D), jnp.float32),
            jax.ShapeDtypeStruct((N, D), jnp.float32),
        ],
    )(X, A_wT, A_b2, B_wT, B_b2)


@functools.cache
def _make_sc_agg():
    @functools.partial(
        pl.kernel,
        out_type=jax.ShapeDtypeStruct((NC, N_ACC, D), jnp.float32),
        mesh=plsc.VectorSubcoreMesh(core_axis_name="c", subcore_axis_name="s"),
        scratch_types=[
            pltpu.VMEM((CMAX, CH), jnp.int32),
            pltpu.VMEM((CMAX, CH), jnp.int32),
            pltpu.VMEM((CH, D), jnp.float32),
            pltpu.VMEM_SHARED((N_ACC, D), jnp.float32),
            pltpu.SemaphoreType.DMA,
        ],
    )
    def _sc_agg(bx_hbm, src_hbm, dst_hbm, zeros_hbm, out_hbm,
                src_v, dst_v, rows_v, acc_sh, sem):
        cid = lax.axis_index("c")
        sid = lax.axis_index("s")
        r0 = sid * RPT
        pltpu.sync_copy(src_hbm.at[cid, sid], src_v)
        pltpu.sync_copy(dst_hbm.at[cid, sid], dst_v)
        pltpu.sync_copy(zeros_hbm.at[pl.ds(r0, RPT)], acc_sh.at[pl.ds(r0, RPT)])
        plsc.subcore_barrier()

        nchunks = jnp.where(cid == 0, C0, C1)

        @pl.loop(0, nchunks)
        def _(j):
            pltpu.async_copy(bx_hbm.at[src_v.at[j]], rows_v, sem).wait()
            pltpu.sync_copy(rows_v, acc_sh.at[dst_v.at[j]], add=True)

        plsc.subcore_barrier()
        pltpu.sync_copy(acc_sh.at[pl.ds(r0, RPT)], out_hbm.at[cid, pl.ds(r0, RPT)])

    return _sc_agg


def _post_body(ax_ref, p0_ref, p1_ref, sn_ref, x_ref, g_ref, b_ref, dep_ref,
               out_ref, sum_ref, sq_ref):
    ph = pl.program_id(0)
    i = pl.program_id(1)
    hpre = (ax_ref[...] + p0_ref[0] + p1_ref[0]) * sn_ref[...]

    @pl.when(jnp.logical_and(ph == 0, i == 0))
    def _():
        sum_ref[...] = jnp.zeros_like(sum_ref)
        sq_ref[...] = jnp.zeros_like(sq_ref)

    @pl.when(ph == 0)
    def _():
        sum_ref[...] += jnp.sum(hpre, axis=0, keepdims=True)
        sq_ref[...] += jnp.sum(hpre * hpre, axis=0, keepdims=True)

    @pl.when(ph == 1)
    def _():
        mean = sum_ref[...] * (1.0 / N)
        var = sq_ref[...] * (1.0 / N) - mean * mean
        scale = lax.rsqrt(var + 1e-5) * g_ref[...]
        h = (hpre - mean) * scale + b_ref[...]
        out_ref[...] = x_ref[...] + jnp.maximum(h, 0.0)


def _post(AX, parts, snorm_n, X, g2, b2, dep):
    return pl.pallas_call(
        _post_body,
        grid=(2, NB),
        in_specs=[
            pl.BlockSpec((ROWS_BLK, D), lambda p, i: (i, 0)),
            pl.BlockSpec((1, ROWS_BLK, D), lambda p, i: (0, i, 0)),
            pl.BlockSpec((1, ROWS_BLK, D), lambda p, i: (1, i, 0)),
            pl.BlockSpec((ROWS_BLK, 1), lambda p, i: (i, 0)),
            pl.BlockSpec((ROWS_BLK, D), lambda p, i: (i, 0)),
            pl.BlockSpec((1, D), lambda p, i: (0, 0)),
            pl.BlockSpec((1, D), lambda p, i: (0, 0)),
            pl.BlockSpec((8, D), lambda p, i: (0, 0)),
        ],
        out_specs=pl.BlockSpec((ROWS_BLK, D), lambda p, i: (i, 0)),
        out_shape=jax.ShapeDtypeStruct((N, D), jnp.float32),
        scratch_shapes=[
            pltpu.VMEM((1, D), jnp.float32),
            pltpu.VMEM((1, D), jnp.float32),
        ],
    )(AX, parts, parts, snorm_n, X, g2, b2, dep)


EB = 8000


def _ex_body(dummy_ref, ex_ref, out_ref):
    out_ref[...] = ex_ref[...]


def _ex_copy(E_X, BX):
    return pl.pallas_call(
        _ex_body,
        grid=(E // EB,),
        in_specs=[
            pl.BlockSpec((8, D), lambda i: (0, 0)),
            pl.BlockSpec((EB, D), lambda i: (i, 0)),
        ],
        out_specs=pl.BlockSpec((EB, D), lambda i: (i, 0)),
        out_shape=jax.ShapeDtypeStruct((E, D), jnp.float32),
    )(BX[:8], E_X)


def kernel(X, edge_index, E_X, snorm_n, snorm_e, A_w, A_b, B_w, B_b, gamma_h, beta_h):
    src = edge_index[0]
    dst = edge_index[1]
    pad = E_PAD - E
    E0 = NS * C0 * CH

    def split(full):
        p0 = full[:E0].reshape(NS, C0, CH)
        p1 = full[E0:].reshape(NS, C1, CH)
        p1 = jnp.pad(p1, ((0, 0), (0, CMAX - C1), (0, 0)))
        return jnp.stack([p0, p1])

    trash = N + jnp.arange(pad, dtype=dst.dtype) % (N_ACC - N)
    src_p = split(jnp.concatenate([src, jnp.zeros((pad,), src.dtype)]))
    dst_p = split(jnp.concatenate([dst, trash]))
    zeros = jnp.zeros((N_ACC, D), jnp.float32)

    AX, BX = _matmuls(X, A_w.T, A_b.reshape(1, D), B_w.T, B_b.reshape(1, D))
    parts = _make_sc_agg()(BX, src_p, dst_p, zeros)
    E_out = _ex_copy(E_X, BX)
    H = _post(AX, parts, snorm_n, X,
              gamma_h.reshape(1, D), beta_h.reshape(1, D), E_out[:8])
    return (H, E_out)
